# baseline (speedup 1.0000x reference)
import jax
import jax.numpy as jnp
from jax.experimental import pallas as pl
from jax.experimental.pallas import tpu as pltpu

SEQ_BLOCK = 512


def _body(t_ref, x_ref, p_ref, o_ref):
    s = pl.program_id(1)
    ns = pl.num_programs(1)

    @pl.when(s < ns - 1)
    def _copy():
        o_ref[...] = x_ref[...]

    @pl.when(s == ns - 1)
    def _prompt_row():
        o_ref[0, 0, :] = p_ref[0, 0, :]


def kernel(x, task_id, prompt):
    B, S, D = x.shape
    n_sb = S // SEQ_BLOCK
    prompt3 = prompt.reshape(prompt.shape[0], 1, D)

    grid_spec = pltpu.PrefetchScalarGridSpec(
        num_scalar_prefetch=1,
        grid=(B, n_sb + 1),
        in_specs=[
            pl.BlockSpec((1, SEQ_BLOCK, D),
                         lambda b, s, t: (b, jnp.minimum(s, n_sb - 1), 0)),
            pl.BlockSpec((1, 1, D), lambda b, s, t: (t[b], 0, 0)),
        ],
        out_specs=pl.BlockSpec((1, SEQ_BLOCK, D), lambda b, s, t: (b, s, 0)),
    )
    out = pl.pallas_call(
        _body,
        grid_spec=grid_spec,
        out_shape=jax.ShapeDtypeStruct((B, S + 1, D), x.dtype),
    )(task_id, x, prompt3)
    return (out, task_id)
